# Initial kernel scaffold; baseline (speedup 1.0000x reference)
import jax
import jax.numpy as jnp
from jax.experimental import pallas as pl


def kernel(x_src, x_target, edge_index, edge_type, target_node_type, rel_W, root_W, root_b):
    raise NotImplementedError("write your pallas kernel here")


# speedup vs baseline: 4.6558x; 4.6558x over previous
import jax
import jax.numpy as jnp
from jax.experimental import pallas as pl

_N = 100000
_C = 128
_NT = 4
_ET = 7
_DROP = 3200000 // 5
_B = 2000


def _rgcn_body(agg_ref, cnt_ref, xt_ref, tnt_ref, relW_ref, rootW_ref,
               rootb_ref, out_ref):
    agg = agg_ref[...]
    cnt = cnt_ref[...]
    acc = jnp.zeros((_B, _C), jnp.float32)
    for t in range(_ET):
        a = agg[:, t * _C:(t + 1) * _C]
        c = jnp.maximum(cnt[:, t:t + 1], 1.0)
        acc = acc + (a / c) @ relW_ref[t]
    xt = xt_ref[...]
    tnt = tnt_ref[...]
    for t in range(_NT):
        m = (tnt == t).astype(jnp.float32)
        acc = acc + m * (xt @ rootW_ref[t] + rootb_ref[t][None, :])
    out_ref[...] = acc


def kernel(x_src, x_target, edge_index, edge_type, target_node_type,
           rel_W, root_W, root_b):
    src = edge_index[0, _DROP:].astype(jnp.int32)
    dst = edge_index[1, _DROP:].astype(jnp.int32)
    et = edge_type[_DROP:].astype(jnp.int32)
    tnt = target_node_type.astype(jnp.int32).reshape(_N, 1)

    flat = dst * _ET + et
    agg = jnp.zeros((_N * _ET, _C), jnp.float32).at[flat].add(
        jnp.take(x_src, src, axis=0))
    cnt = jnp.zeros((_N * _ET,), jnp.float32).at[flat].add(1.0)
    agg = agg.reshape(_N, _ET * _C)
    cnt = cnt.reshape(_N, _ET)

    grid = (_N // _B,)
    out = pl.pallas_call(
        _rgcn_body,
        grid=grid,
        in_specs=[
            pl.BlockSpec((_B, _ET * _C), lambda b: (b, 0)),
            pl.BlockSpec((_B, _ET), lambda b: (b, 0)),
            pl.BlockSpec((_B, _C), lambda b: (b, 0)),
            pl.BlockSpec((_B, 1), lambda b: (b, 0)),
            pl.BlockSpec((_ET, _C, _C), lambda b: (0, 0, 0)),
            pl.BlockSpec((_NT, _C, _C), lambda b: (0, 0, 0)),
            pl.BlockSpec((_NT, _C), lambda b: (0, 0)),
        ],
        out_specs=pl.BlockSpec((_B, _C), lambda b: (b, 0)),
        out_shape=jax.ShapeDtypeStruct((_N, _C), jnp.float32),
    )(agg, cnt, x_target, tnt, rel_W, root_W, root_b)
    return out
